# baseline (speedup 1.0000x reference)
import functools

import jax
import jax.numpy as jnp
from jax import lax
from jax.experimental import pallas as pl
from jax.experimental.pallas import tpu as pltpu
from jax.experimental.pallas import tpu_sc as plsc

N_NODES = 10000
N_EDGES = 160000
NC, NS = 2, 16
NW = NC * NS
CH = 128
CPW = 40
E_PAD = NW * CPW * CH
NB = 8
N_PAD = 10240
RPT = N_PAD // NS
PAD_IDX = N_NODES

def _mesh():
    return plsc.VectorSubcoreMesh(core_axis_name="c", subcore_axis_name="s")


FD = 8
CPT = NW * CPW // NS


def _newton_rsqrt(x):
    yi = jnp.int32(0x5F3759DF) - (plsc.bitcast(x, jnp.int32) >> 1)
    y = plsc.bitcast(yi, jnp.float32)
    for _ in range(3):
        y = y * (1.5 - 0.5 * x * y * y)
    return jnp.where(x > 0, y, 0.0)


@functools.lru_cache(maxsize=None)
def _make_deg_scatter_kernel():
    F = 32

    @functools.partial(
        pl.kernel,
        out_type=[
            jax.ShapeDtypeStruct((NC, N_PAD, F), jnp.float32),
            jax.ShapeDtypeStruct((NC, N_PAD, FD), jnp.float32),
        ],
        mesh=_mesh(),
        scratch_types=(
            [pltpu.VMEM((CPT, CH), jnp.int32)]
            + [pltpu.VMEM((CPW, CH), jnp.int32)] * 2
            + [pltpu.VMEM((CH, FD), jnp.float32)]
            + [pltpu.VMEM((RPT, F), jnp.float32)]
            + [pltpu.VMEM((RPT, FD), jnp.float32)]
            + [pltpu.VMEM((RPT,), jnp.float32)]
            + [pltpu.VMEM((CH, F), jnp.float32)] * NB
            + [pltpu.VMEM_SHARED((N_PAD, F), jnp.float32)] * 2
            + [pltpu.VMEM_SHARED((N_PAD, FD), jnp.float32)]
            + [pltpu.SemaphoreType.DMA] * (2 * NB + 1)
        ),
        compiler_params=pltpu.CompilerParams(use_tc_tiling_on_sc=False,
                                             needs_layout_passes=False),
    )
    def deg_scatter_kernel(y1_hbm, src_hbm, dst_hbm, zeros_hbm, zerosd_hbm,
                           ones_hbm, acc_out, deg_out, *bufs):
        dsidx, sidx, didx, ones_v, ytile, degb, disb = bufs[:7]
        rows = bufs[7:7 + NB]
        acc, u_s, deg = bufs[7 + NB:10 + NB]
        gsem = bufs[10 + NB:10 + 2 * NB]
        ssem = bufs[10 + 2 * NB:10 + 3 * NB]
        dsem = bufs[10 + 3 * NB]
        c = lax.axis_index("c")
        s = lax.axis_index("s")
        w = s * NC + c
        tslice = pl.ds(s * RPT, RPT)
        pltpu.sync_copy(zeros_hbm.at[tslice], acc.at[tslice])
        pltpu.sync_copy(zerosd_hbm.at[tslice], deg.at[tslice])
        pltpu.sync_copy(y1_hbm.at[tslice], ytile)
        pltpu.sync_copy(ones_hbm, ones_v)
        pltpu.sync_copy(src_hbm.at[pl.ds(pl.multiple_of(s * CPT, 8), CPT)],
                        dsidx)
        rbase = pl.multiple_of(w * CPW, 8)
        pltpu.sync_copy(src_hbm.at[pl.ds(rbase, CPW)], sidx)
        pltpu.sync_copy(dst_hbm.at[pl.ds(rbase, CPW)], didx)
        plsc.subcore_barrier()

        def deg_rnd(g, carry):
            ds = [pltpu.async_copy(ones_v, deg.at[dsidx.at[g * NB + b]], dsem,
                                   add=True)
                  for b in range(NB)]
            for d in ds:
                d.wait()
            return carry

        lax.fori_loop(0, CPT // NB, deg_rnd, 0)
        plsc.subcore_barrier()

        pltpu.sync_copy(deg.at[tslice], degb)
        col0 = jnp.zeros((16,), jnp.int32)
        lanes = lax.iota(jnp.int32, 16)

        def dis_grp(g, carry):
            dvals = plsc.load_gather(degb, [g * 16 + lanes, col0])
            disb[pl.ds(g * 16, 16)] = _newton_rsqrt(dvals)
            return carry

        lax.fori_loop(0, RPT // 16, dis_grp, 0)

        def scale_grp(g, carry):
            dvec = disb[pl.ds(g * 16, 16)]
            for b in range(16):
                r = g * 16 + b
                d = dvec[b]
                ytile[r, pl.ds(0, 16)] = ytile[r, pl.ds(0, 16)] * d
                ytile[r, pl.ds(16, 16)] = ytile[r, pl.ds(16, 16)] * d
            return carry

        lax.fori_loop(0, RPT // 16, scale_grp, 0)
        pltpu.sync_copy(ytile, u_s.at[tslice])
        pltpu.sync_copy(deg.at[tslice], deg_out.at[c].at[tslice])
        plsc.subcore_barrier()

        def rnd(g, carry):
            gds = [pltpu.async_copy(u_s.at[sidx.at[g * NB + b]], rows[b],
                                    gsem[b])
                   for b in range(NB)]
            sds = []
            for b in range(NB):
                gds[b].wait()
                sds.append(pltpu.async_copy(rows[b],
                                            acc.at[didx.at[g * NB + b]],
                                            ssem[b], add=True))
            for d in sds:
                d.wait()
            return carry

        lax.fori_loop(0, CPW // NB, rnd, 0)
        plsc.subcore_barrier()
        pltpu.sync_copy(acc.at[tslice], acc_out.at[c].at[tslice])

    return deg_scatter_kernel


@functools.lru_cache(maxsize=None)
def _make_scatter_kernel(F):
    @functools.partial(
        pl.kernel,
        out_type=jax.ShapeDtypeStruct((NC, N_PAD, F), jnp.float32),
        mesh=_mesh(),
        scratch_types=(
            [pltpu.VMEM((CPW, CH), jnp.int32)] * 2
            + [pltpu.VMEM((CH, F), jnp.float32)] * NB
            + [pltpu.VMEM_SHARED((N_PAD, F), jnp.float32)] * 2
            + [pltpu.SemaphoreType.DMA] * (2 * NB)
        ),
        compiler_params=pltpu.CompilerParams(use_tc_tiling_on_sc=False),
    )
    def scatter_kernel(u_hbm, src_hbm, dst_hbm, zeros_hbm, out_hbm,
                       sidx, didx, *bufs):
        rows = bufs[:NB]
        acc = bufs[NB]
        u_s = bufs[NB + 1]
        gsem = bufs[NB + 2:2 * NB + 2]
        ssem = bufs[2 * NB + 2:]
        c = lax.axis_index("c")
        s = lax.axis_index("s")
        w = s * NC + c
        pltpu.sync_copy(zeros_hbm.at[pl.ds(s * RPT, RPT)],
                        acc.at[pl.ds(s * RPT, RPT)])
        pltpu.sync_copy(u_hbm.at[pl.ds(s * RPT, RPT)],
                        u_s.at[pl.ds(s * RPT, RPT)])
        rbase = pl.multiple_of(w * CPW, 8)
        pltpu.sync_copy(src_hbm.at[pl.ds(rbase, CPW)], sidx)
        pltpu.sync_copy(dst_hbm.at[pl.ds(rbase, CPW)], didx)
        plsc.subcore_barrier()

        def rnd(g, carry):
            gds = [pltpu.async_copy(u_s.at[sidx.at[g * NB + b]], rows[b],
                                    gsem[b])
                   for b in range(NB)]
            sds = []
            for b in range(NB):
                gds[b].wait()
                sds.append(pltpu.async_copy(rows[b],
                                            acc.at[didx.at[g * NB + b]],
                                            ssem[b], add=True))
            for d in sds:
                d.wait()
            return carry

        lax.fori_loop(0, CPW // NB, rnd, 0)
        plsc.subcore_barrier()
        pltpu.sync_copy(acc.at[pl.ds(s * RPT, RPT)],
                        out_hbm.at[c].at[pl.ds(s * RPT, RPT)])

    return scatter_kernel


def _t1_body(x_ref, w0_ref, w1_ref, y0_ref, y1_ref):
    xv = x_ref[...]
    y0_ref[...] = jnp.dot(xv, w0_ref[...], preferred_element_type=jnp.float32)
    y1_ref[...] = jnp.dot(xv, w1_ref[...], preferred_element_type=jnp.float32)


def _t2_body(y0_ref, accp_ref, degp_ref, b1_ref, w20_ref, w21_ref,
             z0_ref, u2_ref, dis_ref):
    deg = degp_ref[0, :N_NODES, :1]
    dis = jnp.where(deg > 0, lax.rsqrt(jnp.maximum(deg, 1e-12)), 0.0)
    dis_ref[...] = dis
    tx = dis * (accp_ref[0, :N_NODES] + accp_ref[1, :N_NODES])
    h = jnp.maximum(y0_ref[...] - tx + b1_ref[...], 0.0)
    z0_ref[...] = jnp.dot(h, w20_ref[...], preferred_element_type=jnp.float32)
    u2_ref[...] = dis * jnp.dot(h, w21_ref[...],
                                preferred_element_type=jnp.float32)


def _t3_body(z0_ref, accp_ref, dis_ref, b2_ref, out_ref):
    tx = dis_ref[...] * (accp_ref[0, :N_NODES] + accp_ref[1, :N_NODES])
    o = z0_ref[...] - tx + b2_ref[...]
    m = jnp.max(o, axis=1, keepdims=True)
    e = jnp.exp(o - m)
    out_ref[...] = o - m - jnp.log(jnp.sum(e, axis=1, keepdims=True))


def kernel(x, edge_index, W1_0, W1_1, b1, W2_0, W2_1, b2):
    ei2 = jnp.pad(edge_index, ((0, 0), (0, E_PAD - N_EDGES)),
                  constant_values=PAD_IDX)
    src_pad = ei2[0].reshape(-1, CH)
    dst_pad = ei2[1].reshape(-1, CH)
    ones_ch = jnp.ones((CH, FD), jnp.float32)
    zeros8 = jnp.zeros((N_PAD, FD), jnp.float32)
    zeros32 = jnp.zeros((N_PAD, 32), jnp.float32)
    zeros40 = jnp.zeros((N_PAD, 40), jnp.float32)

    y0, y1 = pl.pallas_call(
        _t1_body,
        out_shape=[
            jax.ShapeDtypeStruct((N_NODES, 32), jnp.float32),
            jax.ShapeDtypeStruct((N_NODES, 32), jnp.float32),
        ],
    )(x, W1_0, W1_1)

    y1p = jnp.pad(y1, ((0, N_PAD - N_NODES), (0, 0)))
    acc1, degp = _make_deg_scatter_kernel()(y1p, src_pad, dst_pad,
                                            zeros32, zeros8, ones_ch)

    z0, u2, dis = pl.pallas_call(
        _t2_body,
        out_shape=[
            jax.ShapeDtypeStruct((N_NODES, 40), jnp.float32),
            jax.ShapeDtypeStruct((N_NODES, 40), jnp.float32),
            jax.ShapeDtypeStruct((N_NODES, 1), jnp.float32),
        ],
    )(y0, acc1, degp, b1, W2_0, W2_1)

    u2p = jnp.pad(u2, ((0, N_PAD - N_NODES), (0, 0)))
    acc2 = _make_scatter_kernel(40)(u2p, src_pad, dst_pad, zeros40)

    out = pl.pallas_call(
        _t3_body,
        out_shape=jax.ShapeDtypeStruct((N_NODES, 40), jnp.float32),
    )(z0, acc2, dis, b2)
    return out


# speedup vs baseline: 1.0777x; 1.0777x over previous
import functools

import jax
import jax.numpy as jnp
from jax import lax
from jax.experimental import pallas as pl
from jax.experimental.pallas import tpu as pltpu
from jax.experimental.pallas import tpu_sc as plsc

N_NODES = 10000
N_EDGES = 160000
NC, NS = 2, 16
NW = NC * NS
CH = 128
CPW = 40
E_PAD = NW * CPW * CH
NB = 8
N_PAD = 10240
RPT = N_PAD // NS
PAD_IDX = N_NODES

def _mesh():
    return plsc.VectorSubcoreMesh(core_axis_name="c", subcore_axis_name="s")


FD = 8


def _newton_rsqrt(x):
    yi = jnp.int32(0x5F3759DF) - (plsc.bitcast(x, jnp.int32) >> 1)
    y = plsc.bitcast(yi, jnp.float32)
    for _ in range(3):
        y = y * (1.5 - 0.5 * x * y * y)
    return jnp.where(x > 0, y, 0.0)


@functools.lru_cache(maxsize=None)
def _make_deg_kernel():
    @functools.partial(
        pl.kernel,
        out_type=jax.ShapeDtypeStruct((NC, N_PAD, FD), jnp.float32),
        mesh=_mesh(),
        scratch_types=(
            [pltpu.VMEM((CPW, CH), jnp.int32)]
            + [pltpu.VMEM((CH, FD), jnp.float32)]
            + [pltpu.VMEM_SHARED((N_PAD, FD), jnp.float32)]
            + [pltpu.SemaphoreType.DMA] * NB
        ),
        compiler_params=pltpu.CompilerParams(use_tc_tiling_on_sc=False),
    )
    def deg_kernel(src_hbm, zerosd_hbm, ones_hbm, deg_out, sidx, ones_v, deg,
                   *sems):
        c = lax.axis_index("c")
        s = lax.axis_index("s")
        w = s * NC + c
        tslice = pl.ds(s * RPT, RPT)
        pltpu.sync_copy(zerosd_hbm.at[tslice], deg.at[tslice])
        pltpu.sync_copy(ones_hbm, ones_v)
        pltpu.sync_copy(src_hbm.at[pl.ds(pl.multiple_of(w * CPW, 8), CPW)],
                        sidx)
        plsc.subcore_barrier()

        def rnd(g, carry):
            ds = [pltpu.async_copy(ones_v, deg.at[sidx.at[g * NB + b]],
                                   sems[b], add=True)
                  for b in range(NB)]
            for d in ds:
                d.wait()
            return carry

        lax.fori_loop(0, CPW // NB, rnd, 0)
        plsc.subcore_barrier()
        pltpu.sync_copy(deg.at[tslice], deg_out.at[c].at[tslice])

    return deg_kernel


@functools.lru_cache(maxsize=None)
def _make_sc1_kernel():
    F = 32

    @functools.partial(
        pl.kernel,
        out_type=jax.ShapeDtypeStruct((NC, N_PAD, F), jnp.float32),
        mesh=_mesh(),
        scratch_types=(
            [pltpu.VMEM((CPW, CH), jnp.int32)] * 2
            + [pltpu.VMEM((RPT, F), jnp.float32)]
            + [pltpu.VMEM((RPT, FD), jnp.float32)] * 2
            + [pltpu.VMEM((RPT,), jnp.float32)]
            + [pltpu.VMEM((CH, F), jnp.float32)] * NB
            + [pltpu.VMEM_SHARED((N_PAD, F), jnp.float32)] * 2
            + [pltpu.SemaphoreType.DMA] * (2 * NB)
        ),
        compiler_params=pltpu.CompilerParams(use_tc_tiling_on_sc=False,
                                             needs_layout_passes=False),
    )
    def sc1_kernel(y1_hbm, src_hbm, dst_hbm, zeros_hbm, deg_hbm, acc_out,
                   *bufs):
        sidx, didx, ytile, degb0, degb1, disb = bufs[:6]
        rows = bufs[6:6 + NB]
        acc, u_s = bufs[6 + NB:8 + NB]
        gsem = bufs[8 + NB:8 + 2 * NB]
        ssem = bufs[8 + 2 * NB:8 + 3 * NB]
        c = lax.axis_index("c")
        s = lax.axis_index("s")
        w = s * NC + c
        tslice = pl.ds(s * RPT, RPT)
        pltpu.sync_copy(zeros_hbm.at[tslice], acc.at[tslice])
        pltpu.sync_copy(y1_hbm.at[tslice], ytile)
        pltpu.sync_copy(deg_hbm.at[0].at[tslice], degb0)
        pltpu.sync_copy(deg_hbm.at[1].at[tslice], degb1)
        rbase = pl.multiple_of(w * CPW, 8)
        pltpu.sync_copy(src_hbm.at[pl.ds(rbase, CPW)], sidx)
        pltpu.sync_copy(dst_hbm.at[pl.ds(rbase, CPW)], didx)

        col0 = jnp.zeros((16,), jnp.int32)
        lanes = lax.iota(jnp.int32, 16)

        def dis_grp(g, carry):
            dvals = (plsc.load_gather(degb0, [g * 16 + lanes, col0])
                     + plsc.load_gather(degb1, [g * 16 + lanes, col0]))
            disb[pl.ds(g * 16, 16)] = _newton_rsqrt(dvals)
            return carry

        lax.fori_loop(0, RPT // 16, dis_grp, 0)

        def scale_grp(g, carry):
            dvec = disb[pl.ds(g * 16, 16)]
            for b in range(16):
                r = g * 16 + b
                d = dvec[b]
                ytile[r, pl.ds(0, 16)] = ytile[r, pl.ds(0, 16)] * d
                ytile[r, pl.ds(16, 16)] = ytile[r, pl.ds(16, 16)] * d
            return carry

        lax.fori_loop(0, RPT // 16, scale_grp, 0)
        pltpu.sync_copy(ytile, u_s.at[tslice])
        plsc.subcore_barrier()

        def rnd(g, carry):
            gds = [pltpu.async_copy(u_s.at[sidx.at[g * NB + b]], rows[b],
                                    gsem[b])
                   for b in range(NB)]
            sds = []
            for b in range(NB):
                gds[b].wait()
                sds.append(pltpu.async_copy(rows[b],
                                            acc.at[didx.at[g * NB + b]],
                                            ssem[b], add=True))
            for d in sds:
                d.wait()
            return carry

        lax.fori_loop(0, CPW // NB, rnd, 0)
        plsc.subcore_barrier()
        pltpu.sync_copy(acc.at[tslice], acc_out.at[c].at[tslice])

    return sc1_kernel


@functools.lru_cache(maxsize=None)
def _make_scatter_kernel(F):
    @functools.partial(
        pl.kernel,
        out_type=jax.ShapeDtypeStruct((NC, N_PAD, F), jnp.float32),
        mesh=_mesh(),
        scratch_types=(
            [pltpu.VMEM((CPW, CH), jnp.int32)] * 2
            + [pltpu.VMEM((CH, F), jnp.float32)] * NB
            + [pltpu.VMEM_SHARED((N_PAD, F), jnp.float32)] * 2
            + [pltpu.SemaphoreType.DMA] * (2 * NB)
        ),
        compiler_params=pltpu.CompilerParams(use_tc_tiling_on_sc=False),
    )
    def scatter_kernel(u_hbm, src_hbm, dst_hbm, zeros_hbm, out_hbm,
                       sidx, didx, *bufs):
        rows = bufs[:NB]
        acc = bufs[NB]
        u_s = bufs[NB + 1]
        gsem = bufs[NB + 2:2 * NB + 2]
        ssem = bufs[2 * NB + 2:]
        c = lax.axis_index("c")
        s = lax.axis_index("s")
        w = s * NC + c
        pltpu.sync_copy(zeros_hbm.at[pl.ds(s * RPT, RPT)],
                        acc.at[pl.ds(s * RPT, RPT)])
        pltpu.sync_copy(u_hbm.at[pl.ds(s * RPT, RPT)],
                        u_s.at[pl.ds(s * RPT, RPT)])
        rbase = pl.multiple_of(w * CPW, 8)
        pltpu.sync_copy(src_hbm.at[pl.ds(rbase, CPW)], sidx)
        pltpu.sync_copy(dst_hbm.at[pl.ds(rbase, CPW)], didx)
        plsc.subcore_barrier()

        def rnd(g, carry):
            gds = [pltpu.async_copy(u_s.at[sidx.at[g * NB + b]], rows[b],
                                    gsem[b])
                   for b in range(NB)]
            sds = []
            for b in range(NB):
                gds[b].wait()
                sds.append(pltpu.async_copy(rows[b],
                                            acc.at[didx.at[g * NB + b]],
                                            ssem[b], add=True))
            for d in sds:
                d.wait()
            return carry

        lax.fori_loop(0, CPW // NB, rnd, 0)
        plsc.subcore_barrier()
        pltpu.sync_copy(acc.at[pl.ds(s * RPT, RPT)],
                        out_hbm.at[c].at[pl.ds(s * RPT, RPT)])

    return scatter_kernel


def _t1_body(x_ref, w0_ref, w1_ref, y0_ref, y1_ref):
    xv = x_ref[...]
    y0_ref[...] = jnp.dot(xv, w0_ref[...], preferred_element_type=jnp.float32)
    y1_ref[...] = jnp.dot(xv, w1_ref[...], preferred_element_type=jnp.float32)


def _t2_body(y0_ref, accp_ref, degp_ref, b1_ref, w20_ref, w21_ref,
             z0_ref, u2_ref, dis_ref):
    deg = degp_ref[0, :N_NODES, :1] + degp_ref[1, :N_NODES, :1]
    dis = jnp.where(deg > 0, lax.rsqrt(jnp.maximum(deg, 1e-12)), 0.0)
    dis_ref[...] = dis
    tx = dis * (accp_ref[0, :N_NODES] + accp_ref[1, :N_NODES])
    h = jnp.maximum(y0_ref[...] - tx + b1_ref[...], 0.0)
    z0_ref[...] = jnp.dot(h, w20_ref[...], preferred_element_type=jnp.float32)
    u2_ref[...] = dis * jnp.dot(h, w21_ref[...],
                                preferred_element_type=jnp.float32)


def _t3_body(z0_ref, accp_ref, dis_ref, b2_ref, out_ref):
    tx = dis_ref[...] * (accp_ref[0, :N_NODES] + accp_ref[1, :N_NODES])
    o = z0_ref[...] - tx + b2_ref[...]
    m = jnp.max(o, axis=1, keepdims=True)
    e = jnp.exp(o - m)
    out_ref[...] = o - m - jnp.log(jnp.sum(e, axis=1, keepdims=True))


def kernel(x, edge_index, W1_0, W1_1, b1, W2_0, W2_1, b2):
    ei2 = jnp.pad(edge_index, ((0, 0), (0, E_PAD - N_EDGES)),
                  constant_values=PAD_IDX)
    src_pad = ei2[0].reshape(-1, CH)
    dst_pad = ei2[1].reshape(-1, CH)
    ones_ch = jnp.ones((CH, FD), jnp.float32)
    zeros8 = jnp.zeros((N_PAD, FD), jnp.float32)
    zeros32 = jnp.zeros((N_PAD, 32), jnp.float32)
    zeros40 = jnp.zeros((N_PAD, 40), jnp.float32)

    degp = _make_deg_kernel()(src_pad, zeros8, ones_ch)

    y0, y1 = pl.pallas_call(
        _t1_body,
        out_shape=[
            jax.ShapeDtypeStruct((N_NODES, 32), jnp.float32),
            jax.ShapeDtypeStruct((N_NODES, 32), jnp.float32),
        ],
    )(x, W1_0, W1_1)

    y1p = jnp.pad(y1, ((0, N_PAD - N_NODES), (0, 0)))
    acc1 = _make_sc1_kernel()(y1p, src_pad, dst_pad, zeros32, degp)

    z0, u2, dis = pl.pallas_call(
        _t2_body,
        out_shape=[
            jax.ShapeDtypeStruct((N_NODES, 40), jnp.float32),
            jax.ShapeDtypeStruct((N_NODES, 40), jnp.float32),
            jax.ShapeDtypeStruct((N_NODES, 1), jnp.float32),
        ],
    )(y0, acc1, degp, b1, W2_0, W2_1)

    u2p = jnp.pad(u2, ((0, N_PAD - N_NODES), (0, 0)))
    acc2 = _make_scatter_kernel(40)(u2p, src_pad, dst_pad, zeros40)

    out = pl.pallas_call(
        _t3_body,
        out_shape=jax.ShapeDtypeStruct((N_NODES, 40), jnp.float32),
    )(z0, acc2, dis, b2)
    return out
